# baseline (speedup 1.0000x reference)
import functools

import jax
import jax.numpy as jnp
from jax import lax
from jax.experimental import pallas as pl
from jax.experimental.pallas import tpu as pltpu
from jax.experimental.pallas import tpu_sc as plsc

NC = 2
NS = 16
NW = NC * NS
CHUNK = 128
NCHUNK = 80
HALF = NCHUNK // 2
ROW_BLK = 400


def _table_matmul_body(x_ref, w_ref, out_ref, *, R):
    x = x_ref[...]
    for r in range(R):
        out_ref[r] = jnp.dot(x, w_ref[r], preferred_element_type=jnp.float32)


def _prep_body(ei_ref, et_ref, fidx_ref, dst_ref, *, N, R, N_pad, n_pad_rows):
    src = ei_ref[0]
    dstv = ei_ref[1]
    fidx_real = et_ref[...] * N + src
    k = (lax.broadcasted_iota(jnp.int32, (n_pad_rows, 128), 0) * 128
         + lax.broadcasted_iota(jnp.int32, (n_pad_rows, 128), 1))
    fidx_pad = (k % R) * N + k
    dst_pad = N + k % (N_pad - N)
    fidx_ref[...] = jnp.concatenate([fidx_real, fidx_pad], axis=0)
    dst_ref[...] = jnp.concatenate([dstv, dst_pad], axis=0)


def _make_edge_kernel(NR, D, N_pad):
    rows = N_pad // NS
    mesh = plsc.VectorSubcoreMesh(core_axis_name="c", subcore_axis_name="s")

    @functools.partial(
        pl.kernel,
        out_type=jax.ShapeDtypeStruct((NC, N_pad, D), jnp.float32),
        mesh=mesh,
        scratch_types=[
            pltpu.VMEM((HALF, CHUNK), jnp.int32),
            pltpu.VMEM((HALF, CHUNK), jnp.int32),
            pltpu.VMEM((CHUNK, D), jnp.float32),
            pltpu.VMEM((CHUNK, D), jnp.float32),
            pltpu.VMEM_SHARED((N_pad, D), jnp.float32),
            pltpu.SemaphoreType.DMA,
            pltpu.SemaphoreType.DMA,
        ],
    )
    def edge_kernel(xw_hbm, fidx_hbm, dst_hbm, zeros_hbm, out_hbm,
                    fidx_v, dst_v, r0, r1, agg, sem0, sem1):
        c = lax.axis_index("c")
        s = lax.axis_index("s")
        wid = c * NS + s
        z0 = s * rows
        pltpu.sync_copy(zeros_hbm.at[pl.ds(z0, rows)], agg.at[pl.ds(z0, rows)])
        plsc.subcore_barrier()

        for half in range(NCHUNK // HALF):
            base = half * HALF
            pltpu.sync_copy(fidx_hbm.at[wid, pl.ds(base, HALF)], fidx_v)
            pltpu.sync_copy(dst_hbm.at[wid, pl.ds(base, HALF)], dst_v)

            pltpu.async_copy(xw_hbm.at[fidx_v.at[0]], r0, sem0)
            pltpu.async_copy(xw_hbm.at[fidx_v.at[1]], r1, sem1)

            def body(j, carry):
                c0 = 2 * j
                pltpu.make_async_copy(xw_hbm.at[fidx_v.at[c0]], r0, sem0).wait()
                pltpu.sync_copy(r0, agg.at[dst_v.at[c0]], add=True)
                pltpu.async_copy(xw_hbm.at[fidx_v.at[c0 + 2]], r0, sem0)
                pltpu.make_async_copy(
                    xw_hbm.at[fidx_v.at[c0 + 1]], r1, sem1).wait()
                pltpu.sync_copy(r1, agg.at[dst_v.at[c0 + 1]], add=True)
                pltpu.async_copy(xw_hbm.at[fidx_v.at[c0 + 3]], r1, sem1)
                return carry

            lax.fori_loop(0, HALF // 2 - 1, body, 0)
            last = HALF - 2
            pltpu.make_async_copy(xw_hbm.at[fidx_v.at[last]], r0, sem0).wait()
            pltpu.sync_copy(r0, agg.at[dst_v.at[last]], add=True)
            pltpu.make_async_copy(xw_hbm.at[fidx_v.at[last + 1]], r1, sem1).wait()
            pltpu.sync_copy(r1, agg.at[dst_v.at[last + 1]], add=True)

        plsc.subcore_barrier()
        pltpu.sync_copy(agg.at[pl.ds(z0, rows)], out_hbm.at[c, pl.ds(z0, rows)])

    return edge_kernel


def _pre_tail_body(x_ref, h1_ref, h2_ref, lw_ref, wg_ref, rb_ref, gb_ref,
                   xlw_ref, gh_ref, *, D):
    xlw_ref[...] = (jnp.dot(x_ref[...], lw_ref[...],
                            preferred_element_type=jnp.float32) + rb_ref[...])
    wg = wg_ref[...]
    gh_ref[...] = (jnp.dot(h1_ref[...], wg[D:2 * D],
                           preferred_element_type=jnp.float32)
                   + jnp.dot(h2_ref[...], wg[2 * D:],
                             preferred_element_type=jnp.float32)
                   + gb_ref[...])


def _tail_body(p0_ref, p1_ref, xlw_ref, ghl_ref, h1_ref, h2_ref, wg0_ref,
               wih_ref, whh_ref, bih_ref, bhh_ref,
               h_ref, gate_ref, *, D):
    h1 = h1_ref[...]
    h2 = h2_ref[...]
    spatial = p0_ref[0] + p1_ref[0] + xlw_ref[...]
    gate_lin = (jnp.dot(spatial, wg0_ref[...],
                        preferred_element_type=jnp.float32)
                + ghl_ref[...])
    gate = jax.nn.sigmoid(gate_lin)
    fused = gate * h1 + (1.0 - gate) * h2
    gi = jnp.dot(spatial, wih_ref[...],
                 preferred_element_type=jnp.float32) + bih_ref[...]
    gh = jnp.dot(fused, whh_ref[...],
                 preferred_element_type=jnp.float32) + bhh_ref[...]
    r = jax.nn.sigmoid(gi[:, :D] + gh[:, :D])
    z = jax.nn.sigmoid(gi[:, D:2 * D] + gh[:, D:2 * D])
    n = jnp.tanh(gi[:, 2 * D:] + r * gh[:, 2 * D:])
    h_t = (1.0 - z) * n + z * fused
    h_ref[...] = jnp.maximum(h_t, 0.0)
    gate_ref[...] = gate


def kernel(x, prev_h_t1, prev_h_t2, edge_index, etype, rel_weight,
           loop_weight, rgcn_bias, gate_W, gate_b, W_ih, W_hh, b_ih, b_hh):
    N, D = x.shape
    R = rel_weight.shape[0]
    E = etype.shape[0]
    N_pad = ((N + 1) + NS * 8 - 1) // (NS * 8) * (NS * 8)
    E_pad = NW * NCHUNK * CHUNK
    pad = E_pad - E

    n_blk = N // ROW_BLK
    xw = pl.pallas_call(
        functools.partial(_table_matmul_body, R=R),
        grid=(n_blk,),
        in_specs=[pl.BlockSpec((ROW_BLK, D), lambda i: (i, 0)),
                  pl.BlockSpec((R, D, D), lambda i: (0, 0, 0))],
        out_specs=pl.BlockSpec((R, ROW_BLK, D), lambda i: (0, i, 0)),
        out_shape=jax.ShapeDtypeStruct((R, N, D), jnp.float32),
    )(x, rel_weight)

    fidx, dst_p = pl.pallas_call(
        functools.partial(_prep_body, N=N, R=R, N_pad=N_pad,
                          n_pad_rows=pad // 128),
        out_shape=[jax.ShapeDtypeStruct((E_pad // 128, 128), jnp.int32),
                   jax.ShapeDtypeStruct((E_pad // 128, 128), jnp.int32)],
    )(edge_index.reshape(2, E // 128, 128), etype.reshape(E // 128, 128))

    edge_kernel = _make_edge_kernel(N * R, D, N_pad)
    partials = edge_kernel(
        xw.reshape(R * N, D),
        fidx.reshape(NW, NCHUNK, CHUNK),
        dst_p.reshape(NW, NCHUNK, CHUNK),
        jnp.zeros((N_pad, D), jnp.float32),
    )

    wg_t = gate_W.T
    wih_t = W_ih.T
    whh_t = W_hh.T
    row = lambda i: (i, 0)
    fixed2 = lambda i: (0, 0)
    xlw, gate_hl = pl.pallas_call(
        functools.partial(_pre_tail_body, D=D),
        grid=(n_blk,),
        in_specs=[
            pl.BlockSpec((ROW_BLK, D), row),
            pl.BlockSpec((ROW_BLK, D), row),
            pl.BlockSpec((ROW_BLK, D), row),
            pl.BlockSpec((D, D), fixed2),
            pl.BlockSpec((3 * D, D), fixed2),
            pl.BlockSpec((1, D), fixed2),
            pl.BlockSpec((1, D), fixed2),
        ],
        out_specs=[pl.BlockSpec((ROW_BLK, D), row),
                   pl.BlockSpec((ROW_BLK, D), row)],
        out_shape=[jax.ShapeDtypeStruct((N, D), jnp.float32),
                   jax.ShapeDtypeStruct((N, D), jnp.float32)],
    )(x, prev_h_t1, prev_h_t2, loop_weight, wg_t,
      rgcn_bias.reshape(1, D), gate_b.reshape(1, D))

    h_t, gate = pl.pallas_call(
        functools.partial(_tail_body, D=D),
        grid=(n_blk,),
        in_specs=[
            pl.BlockSpec((1, ROW_BLK, D), lambda i: (0, i, 0)),
            pl.BlockSpec((1, ROW_BLK, D), lambda i: (1, i, 0)),
            pl.BlockSpec((ROW_BLK, D), row),
            pl.BlockSpec((ROW_BLK, D), row),
            pl.BlockSpec((ROW_BLK, D), row),
            pl.BlockSpec((ROW_BLK, D), row),
            pl.BlockSpec((D, D), fixed2),
            pl.BlockSpec((D, 3 * D), fixed2),
            pl.BlockSpec((D, 3 * D), fixed2),
            pl.BlockSpec((1, 3 * D), fixed2),
            pl.BlockSpec((1, 3 * D), fixed2),
        ],
        out_specs=[pl.BlockSpec((ROW_BLK, D), row),
                   pl.BlockSpec((ROW_BLK, D), row)],
        out_shape=[jax.ShapeDtypeStruct((N, D), jnp.float32),
                   jax.ShapeDtypeStruct((N, D), jnp.float32)],
    )(partials, partials, xlw, gate_hl, prev_h_t1, prev_h_t2, wg_t[:D],
      wih_t, whh_t, b_ih.reshape(1, 3 * D), b_hh.reshape(1, 3 * D))
    return (h_t, gate)


# speedup vs baseline: 1.0966x; 1.0966x over previous
import functools

import jax
import jax.numpy as jnp
from jax import lax
from jax.experimental import pallas as pl
from jax.experimental.pallas import tpu as pltpu
from jax.experimental.pallas import tpu_sc as plsc

NC = 2
NS = 16
NW = NC * NS
CHUNK = 128
NCHUNK = 80
HALF = NCHUNK // 2
ROW_BLK = 1000


def _table_matmul_body(x_ref, w_ref, out_ref, *, R):
    x = x_ref[...]
    for r in range(R):
        out_ref[r] = jnp.dot(x, w_ref[r], preferred_element_type=jnp.float32)


def _prep_body(ei_ref, et_ref, fidx_ref, dst_ref, *, N, R, N_pad, n_pad_rows):
    src = ei_ref[0]
    dstv = ei_ref[1]
    fidx_real = et_ref[...] * N + src
    k = (lax.broadcasted_iota(jnp.int32, (n_pad_rows, 128), 0) * 128
         + lax.broadcasted_iota(jnp.int32, (n_pad_rows, 128), 1))
    fidx_pad = (k % R) * N + k
    dst_pad = N + k % (N_pad - N)
    fidx_ref[...] = jnp.concatenate([fidx_real, fidx_pad], axis=0)
    dst_ref[...] = jnp.concatenate([dstv, dst_pad], axis=0)


def _make_edge_kernel(NR, D, N_pad):
    rows = N_pad // NS
    mesh = plsc.VectorSubcoreMesh(core_axis_name="c", subcore_axis_name="s")

    @functools.partial(
        pl.kernel,
        out_type=jax.ShapeDtypeStruct((NC, N_pad, D), jnp.float32),
        mesh=mesh,
        scratch_types=[
            pltpu.VMEM((HALF, CHUNK), jnp.int32),
            pltpu.VMEM((HALF, CHUNK), jnp.int32),
            pltpu.VMEM((CHUNK, D), jnp.float32),
            pltpu.VMEM((CHUNK, D), jnp.float32),
            pltpu.VMEM_SHARED((N_pad, D), jnp.float32),
            pltpu.SemaphoreType.DMA,
            pltpu.SemaphoreType.DMA,
        ],
    )
    def edge_kernel(xw_hbm, fidx_hbm, dst_hbm, zeros_hbm, out_hbm,
                    fidx_v, dst_v, r0, r1, agg, sem0, sem1):
        c = lax.axis_index("c")
        s = lax.axis_index("s")
        wid = c * NS + s
        z0 = s * rows
        pltpu.sync_copy(zeros_hbm.at[pl.ds(z0, rows)], agg.at[pl.ds(z0, rows)])
        plsc.subcore_barrier()

        for half in range(NCHUNK // HALF):
            base = half * HALF
            pltpu.sync_copy(fidx_hbm.at[wid, pl.ds(base, HALF)], fidx_v)
            pltpu.sync_copy(dst_hbm.at[wid, pl.ds(base, HALF)], dst_v)

            pltpu.async_copy(xw_hbm.at[fidx_v.at[0]], r0, sem0)
            pltpu.async_copy(xw_hbm.at[fidx_v.at[1]], r1, sem1)

            def body(j, carry):
                c0 = 2 * j
                pltpu.make_async_copy(xw_hbm.at[fidx_v.at[c0]], r0, sem0).wait()
                pltpu.sync_copy(r0, agg.at[dst_v.at[c0]], add=True)
                pltpu.async_copy(xw_hbm.at[fidx_v.at[c0 + 2]], r0, sem0)
                pltpu.make_async_copy(
                    xw_hbm.at[fidx_v.at[c0 + 1]], r1, sem1).wait()
                pltpu.sync_copy(r1, agg.at[dst_v.at[c0 + 1]], add=True)
                pltpu.async_copy(xw_hbm.at[fidx_v.at[c0 + 3]], r1, sem1)
                return carry

            lax.fori_loop(0, HALF // 2 - 1, body, 0)
            last = HALF - 2
            pltpu.make_async_copy(xw_hbm.at[fidx_v.at[last]], r0, sem0).wait()
            pltpu.sync_copy(r0, agg.at[dst_v.at[last]], add=True)
            pltpu.make_async_copy(xw_hbm.at[fidx_v.at[last + 1]], r1, sem1).wait()
            pltpu.sync_copy(r1, agg.at[dst_v.at[last + 1]], add=True)

        plsc.subcore_barrier()
        pltpu.sync_copy(agg.at[pl.ds(z0, rows)], out_hbm.at[c, pl.ds(z0, rows)])

    return edge_kernel


def _tail_body(p0_ref, p1_ref, x_ref, h1_ref, h2_ref, lw_ref, wg_ref,
               wih_ref, whh_ref, rb_ref, gb_ref, bih_ref, bhh_ref,
               h_ref, gate_ref, *, D):
    h1 = h1_ref[...]
    h2 = h2_ref[...]
    spatial = (p0_ref[0] + p1_ref[0]
               + jnp.dot(x_ref[...], lw_ref[...],
                         preferred_element_type=jnp.float32)
               + rb_ref[...])
    wg = wg_ref[...]
    gate_lin = (jnp.dot(spatial, wg[:D], preferred_element_type=jnp.float32)
                + jnp.dot(h1, wg[D:2 * D], preferred_element_type=jnp.float32)
                + jnp.dot(h2, wg[2 * D:], preferred_element_type=jnp.float32)
                + gb_ref[...])
    gate = jax.nn.sigmoid(gate_lin)
    fused = gate * h1 + (1.0 - gate) * h2
    gi = jnp.dot(spatial, wih_ref[...],
                 preferred_element_type=jnp.float32) + bih_ref[...]
    gh = jnp.dot(fused, whh_ref[...],
                 preferred_element_type=jnp.float32) + bhh_ref[...]
    r = jax.nn.sigmoid(gi[:, :D] + gh[:, :D])
    z = jax.nn.sigmoid(gi[:, D:2 * D] + gh[:, D:2 * D])
    n = jnp.tanh(gi[:, 2 * D:] + r * gh[:, 2 * D:])
    h_t = (1.0 - z) * n + z * fused
    h_ref[...] = jnp.maximum(h_t, 0.0)
    gate_ref[...] = gate


def kernel(x, prev_h_t1, prev_h_t2, edge_index, etype, rel_weight,
           loop_weight, rgcn_bias, gate_W, gate_b, W_ih, W_hh, b_ih, b_hh):
    N, D = x.shape
    R = rel_weight.shape[0]
    E = etype.shape[0]
    N_pad = ((N + 1) + NS * 8 - 1) // (NS * 8) * (NS * 8)
    E_pad = NW * NCHUNK * CHUNK
    pad = E_pad - E

    n_blk = N // ROW_BLK
    xw = pl.pallas_call(
        functools.partial(_table_matmul_body, R=R),
        grid=(n_blk,),
        in_specs=[pl.BlockSpec((ROW_BLK, D), lambda i: (i, 0)),
                  pl.BlockSpec((R, D, D), lambda i: (0, 0, 0))],
        out_specs=pl.BlockSpec((R, ROW_BLK, D), lambda i: (0, i, 0)),
        out_shape=jax.ShapeDtypeStruct((R, N, D), jnp.float32),
    )(x, rel_weight)

    fidx, dst_p = pl.pallas_call(
        functools.partial(_prep_body, N=N, R=R, N_pad=N_pad,
                          n_pad_rows=pad // 128),
        out_shape=[jax.ShapeDtypeStruct((E_pad // 128, 128), jnp.int32),
                   jax.ShapeDtypeStruct((E_pad // 128, 128), jnp.int32)],
    )(edge_index.reshape(2, E // 128, 128), etype.reshape(E // 128, 128))

    edge_kernel = _make_edge_kernel(N * R, D, N_pad)
    partials = edge_kernel(
        xw.reshape(R * N, D),
        fidx.reshape(NW, NCHUNK, CHUNK),
        dst_p.reshape(NW, NCHUNK, CHUNK),
        jnp.zeros((N_pad, D), jnp.float32),
    )

    wg_t = gate_W.T
    wih_t = W_ih.T
    whh_t = W_hh.T
    row = lambda i: (i, 0)
    fixed2 = lambda i: (0, 0)
    h_t, gate = pl.pallas_call(
        functools.partial(_tail_body, D=D),
        grid=(n_blk,),
        in_specs=[
            pl.BlockSpec((1, ROW_BLK, D), lambda i: (0, i, 0)),
            pl.BlockSpec((1, ROW_BLK, D), lambda i: (1, i, 0)),
            pl.BlockSpec((ROW_BLK, D), row),
            pl.BlockSpec((ROW_BLK, D), row),
            pl.BlockSpec((ROW_BLK, D), row),
            pl.BlockSpec((D, D), fixed2),
            pl.BlockSpec((3 * D, D), fixed2),
            pl.BlockSpec((D, 3 * D), fixed2),
            pl.BlockSpec((D, 3 * D), fixed2),
            pl.BlockSpec((1, D), fixed2),
            pl.BlockSpec((1, D), fixed2),
            pl.BlockSpec((1, 3 * D), fixed2),
            pl.BlockSpec((1, 3 * D), fixed2),
        ],
        out_specs=[pl.BlockSpec((ROW_BLK, D), row),
                   pl.BlockSpec((ROW_BLK, D), row)],
        out_shape=[jax.ShapeDtypeStruct((N, D), jnp.float32),
                   jax.ShapeDtypeStruct((N, D), jnp.float32)],
    )(partials, partials, x, prev_h_t1, prev_h_t2, loop_weight, wg_t,
      wih_t, whh_t, rgcn_bias.reshape(1, D), gate_b.reshape(1, D),
      b_ih.reshape(1, 3 * D), b_hh.reshape(1, 3 * D))
    return (h_t, gate)


# speedup vs baseline: 1.1398x; 1.0394x over previous
import functools

import jax
import jax.numpy as jnp
from jax import lax
from jax.experimental import pallas as pl
from jax.experimental.pallas import tpu as pltpu
from jax.experimental.pallas import tpu_sc as plsc

NC = 2
NS = 16
NW = NC * NS
CHUNK = 128
NCHUNK = 80
HALF = NCHUNK // 2
ROW_BLK = 2000


def _table_matmul_body(x_ref, w_ref, out_ref, *, R):
    x = x_ref[...]
    for r in range(R):
        out_ref[r] = jnp.dot(x, w_ref[r], preferred_element_type=jnp.float32)


def _prep_body(ei_ref, et_ref, fidx_ref, dst_ref, *, N, R, N_pad, n_pad_rows):
    src = ei_ref[0]
    dstv = ei_ref[1]
    fidx_real = et_ref[...] * N + src
    k = (lax.broadcasted_iota(jnp.int32, (n_pad_rows, 128), 0) * 128
         + lax.broadcasted_iota(jnp.int32, (n_pad_rows, 128), 1))
    fidx_pad = (k % R) * N + k
    dst_pad = N + k % (N_pad - N)
    fidx_ref[...] = jnp.concatenate([fidx_real, fidx_pad], axis=0)
    dst_ref[...] = jnp.concatenate([dstv, dst_pad], axis=0)


def _make_edge_kernel(NR, D, N_pad):
    rows = N_pad // NS
    mesh = plsc.VectorSubcoreMesh(core_axis_name="c", subcore_axis_name="s")

    @functools.partial(
        pl.kernel,
        out_type=jax.ShapeDtypeStruct((NC, N_pad, D), jnp.float32),
        mesh=mesh,
        scratch_types=[
            pltpu.VMEM((HALF, CHUNK), jnp.int32),
            pltpu.VMEM((HALF, CHUNK), jnp.int32),
            pltpu.VMEM((CHUNK, D), jnp.float32),
            pltpu.VMEM((CHUNK, D), jnp.float32),
            pltpu.VMEM_SHARED((N_pad, D), jnp.float32),
            pltpu.SemaphoreType.DMA,
            pltpu.SemaphoreType.DMA,
        ],
    )
    def edge_kernel(xw_hbm, fidx_hbm, dst_hbm, zeros_hbm, out_hbm,
                    fidx_v, dst_v, r0, r1, agg, sem0, sem1):
        c = lax.axis_index("c")
        s = lax.axis_index("s")
        wid = c * NS + s
        z0 = s * rows
        pltpu.sync_copy(zeros_hbm.at[pl.ds(z0, rows)], agg.at[pl.ds(z0, rows)])
        plsc.subcore_barrier()

        for half in range(NCHUNK // HALF):
            base = half * HALF
            pltpu.sync_copy(fidx_hbm.at[wid, pl.ds(base, HALF)], fidx_v)
            pltpu.sync_copy(dst_hbm.at[wid, pl.ds(base, HALF)], dst_v)

            pltpu.async_copy(xw_hbm.at[fidx_v.at[0]], r0, sem0)
            pltpu.async_copy(xw_hbm.at[fidx_v.at[1]], r1, sem1)

            def body(j, carry):
                c0 = 2 * j
                pltpu.make_async_copy(xw_hbm.at[fidx_v.at[c0]], r0, sem0).wait()
                pltpu.sync_copy(r0, agg.at[dst_v.at[c0]], add=True)
                pltpu.async_copy(xw_hbm.at[fidx_v.at[c0 + 2]], r0, sem0)
                pltpu.make_async_copy(
                    xw_hbm.at[fidx_v.at[c0 + 1]], r1, sem1).wait()
                pltpu.sync_copy(r1, agg.at[dst_v.at[c0 + 1]], add=True)
                pltpu.async_copy(xw_hbm.at[fidx_v.at[c0 + 3]], r1, sem1)
                return carry

            lax.fori_loop(0, HALF // 2 - 1, body, 0)
            last = HALF - 2
            pltpu.make_async_copy(xw_hbm.at[fidx_v.at[last]], r0, sem0).wait()
            pltpu.sync_copy(r0, agg.at[dst_v.at[last]], add=True)
            pltpu.make_async_copy(xw_hbm.at[fidx_v.at[last + 1]], r1, sem1).wait()
            pltpu.sync_copy(r1, agg.at[dst_v.at[last + 1]], add=True)

        plsc.subcore_barrier()
        pltpu.sync_copy(agg.at[pl.ds(z0, rows)], out_hbm.at[c, pl.ds(z0, rows)])

    return edge_kernel


def _tail_body(p0_ref, p1_ref, x_ref, h1_ref, h2_ref, lw_ref, wg_ref,
               wih_ref, whh_ref, rb_ref, gb_ref, bih_ref, bhh_ref,
               h_ref, gate_ref, *, D):
    h1 = h1_ref[...]
    h2 = h2_ref[...]
    spatial = (p0_ref[0] + p1_ref[0]
               + jnp.dot(x_ref[...], lw_ref[...],
                         preferred_element_type=jnp.float32)
               + rb_ref[...])
    wg = wg_ref[...]
    gate_lin = (jnp.dot(spatial, wg[:D], preferred_element_type=jnp.float32)
                + jnp.dot(h1, wg[D:2 * D], preferred_element_type=jnp.float32)
                + jnp.dot(h2, wg[2 * D:], preferred_element_type=jnp.float32)
                + gb_ref[...])
    gate = jax.nn.sigmoid(gate_lin)
    fused = gate * h1 + (1.0 - gate) * h2
    gi = jnp.dot(spatial, wih_ref[...],
                 preferred_element_type=jnp.float32) + bih_ref[...]
    gh = jnp.dot(fused, whh_ref[...],
                 preferred_element_type=jnp.float32) + bhh_ref[...]
    r = jax.nn.sigmoid(gi[:, :D] + gh[:, :D])
    z = jax.nn.sigmoid(gi[:, D:2 * D] + gh[:, D:2 * D])
    n = jnp.tanh(gi[:, 2 * D:] + r * gh[:, 2 * D:])
    h_t = (1.0 - z) * n + z * fused
    h_ref[...] = jnp.maximum(h_t, 0.0)
    gate_ref[...] = gate


def kernel(x, prev_h_t1, prev_h_t2, edge_index, etype, rel_weight,
           loop_weight, rgcn_bias, gate_W, gate_b, W_ih, W_hh, b_ih, b_hh):
    N, D = x.shape
    R = rel_weight.shape[0]
    E = etype.shape[0]
    N_pad = ((N + 1) + NS * 8 - 1) // (NS * 8) * (NS * 8)
    E_pad = NW * NCHUNK * CHUNK
    pad = E_pad - E

    n_blk = N // ROW_BLK
    xw = pl.pallas_call(
        functools.partial(_table_matmul_body, R=R),
        grid=(n_blk,),
        in_specs=[pl.BlockSpec((ROW_BLK, D), lambda i: (i, 0)),
                  pl.BlockSpec((R, D, D), lambda i: (0, 0, 0))],
        out_specs=pl.BlockSpec((R, ROW_BLK, D), lambda i: (0, i, 0)),
        out_shape=jax.ShapeDtypeStruct((R, N, D), jnp.float32),
    )(x, rel_weight)

    fidx, dst_p = pl.pallas_call(
        functools.partial(_prep_body, N=N, R=R, N_pad=N_pad,
                          n_pad_rows=pad // 128),
        out_shape=[jax.ShapeDtypeStruct((E_pad // 128, 128), jnp.int32),
                   jax.ShapeDtypeStruct((E_pad // 128, 128), jnp.int32)],
    )(edge_index.reshape(2, E // 128, 128), etype.reshape(E // 128, 128))

    edge_kernel = _make_edge_kernel(N * R, D, N_pad)
    partials = edge_kernel(
        xw.reshape(R * N, D),
        fidx.reshape(NW, NCHUNK, CHUNK),
        dst_p.reshape(NW, NCHUNK, CHUNK),
        jnp.zeros((N_pad, D), jnp.float32),
    )

    wg_t = gate_W.T
    wih_t = W_ih.T
    whh_t = W_hh.T
    row = lambda i: (i, 0)
    fixed2 = lambda i: (0, 0)
    h_t, gate = pl.pallas_call(
        functools.partial(_tail_body, D=D),
        grid=(n_blk,),
        in_specs=[
            pl.BlockSpec((1, ROW_BLK, D), lambda i: (0, i, 0)),
            pl.BlockSpec((1, ROW_BLK, D), lambda i: (1, i, 0)),
            pl.BlockSpec((ROW_BLK, D), row),
            pl.BlockSpec((ROW_BLK, D), row),
            pl.BlockSpec((ROW_BLK, D), row),
            pl.BlockSpec((D, D), fixed2),
            pl.BlockSpec((3 * D, D), fixed2),
            pl.BlockSpec((D, 3 * D), fixed2),
            pl.BlockSpec((D, 3 * D), fixed2),
            pl.BlockSpec((1, D), fixed2),
            pl.BlockSpec((1, D), fixed2),
            pl.BlockSpec((1, 3 * D), fixed2),
            pl.BlockSpec((1, 3 * D), fixed2),
        ],
        out_specs=[pl.BlockSpec((ROW_BLK, D), row),
                   pl.BlockSpec((ROW_BLK, D), row)],
        out_shape=[jax.ShapeDtypeStruct((N, D), jnp.float32),
                   jax.ShapeDtypeStruct((N, D), jnp.float32)],
    )(partials, partials, x, prev_h_t1, prev_h_t2, loop_weight, wg_t,
      wih_t, whh_t, rgcn_bias.reshape(1, D), gate_b.reshape(1, D),
      b_ih.reshape(1, 3 * D), b_hh.reshape(1, 3 * D))
    return (h_t, gate)


# speedup vs baseline: 1.1733x; 1.0294x over previous
import functools

import jax
import jax.numpy as jnp
from jax import lax
from jax.experimental import pallas as pl
from jax.experimental.pallas import tpu as pltpu
from jax.experimental.pallas import tpu_sc as plsc

NC = 2
NS = 16
NW = NC * NS
CHUNK = 128
NCHUNK = 80
HALF = NCHUNK // 2
ROW_BLK = 2000


def _table_matmul_body(x_ref, w_ref, out_ref, *, R):
    x = x_ref[...]
    for r in range(R):
        out_ref[r] = jnp.dot(x, w_ref[r], preferred_element_type=jnp.float32)


def _prep_body(ei_ref, et_ref, fidx_ref, dst_ref, *, N, R, N_pad, n_pad_rows):
    src = ei_ref[0]
    dstv = ei_ref[1]
    fidx_real = et_ref[...] * N + src
    k = (lax.broadcasted_iota(jnp.int32, (n_pad_rows, 128), 0) * 128
         + lax.broadcasted_iota(jnp.int32, (n_pad_rows, 128), 1))
    fidx_pad = (k % R) * N + k
    dst_pad = N + k % (N_pad - N)
    fidx_ref[...] = jnp.concatenate([fidx_real, fidx_pad], axis=0)
    dst_ref[...] = jnp.concatenate([dstv, dst_pad], axis=0)


def _make_edge_kernel(NR, D, N_pad):
    rows = N_pad // NS
    mesh = plsc.VectorSubcoreMesh(core_axis_name="c", subcore_axis_name="s")

    @functools.partial(
        pl.kernel,
        out_type=jax.ShapeDtypeStruct((NC, N_pad, D), jnp.float32),
        mesh=mesh,
        scratch_types=[
            pltpu.VMEM((HALF, CHUNK), jnp.int32),
            pltpu.VMEM((HALF, CHUNK), jnp.int32),
            pltpu.VMEM((CHUNK, D), jnp.float32),
            pltpu.VMEM((CHUNK, D), jnp.float32),
            pltpu.VMEM_SHARED((N_pad, D), jnp.float32),
            pltpu.SemaphoreType.DMA,
            pltpu.SemaphoreType.DMA,
        ],
    )
    def edge_kernel(xw_hbm, fidx_hbm, dst_hbm, out_hbm,
                    fidx_v, dst_v, r0, r1, agg, sem0, sem1):
        c = lax.axis_index("c")
        s = lax.axis_index("s")
        wid = c * NS + s
        z0 = s * rows

        def zrow(i, carry):
            for kk in range(D // 16):
                r0[i, pl.ds(kk * 16, 16)] = jnp.zeros((16,), jnp.float32)
            return carry

        lax.fori_loop(0, CHUNK, zrow, 0)
        nfull = rows // CHUNK
        for t in range(nfull):
            pltpu.sync_copy(r0, agg.at[pl.ds(z0 + t * CHUNK, CHUNK)])
        rem = rows - nfull * CHUNK
        if rem:
            pltpu.sync_copy(r0.at[pl.ds(0, rem)],
                            agg.at[pl.ds(z0 + nfull * CHUNK, rem)])
        plsc.subcore_barrier()

        for half in range(NCHUNK // HALF):
            base = half * HALF
            pltpu.sync_copy(fidx_hbm.at[wid, pl.ds(base, HALF)], fidx_v)
            pltpu.sync_copy(dst_hbm.at[wid, pl.ds(base, HALF)], dst_v)

            pltpu.async_copy(xw_hbm.at[fidx_v.at[0]], r0, sem0)
            pltpu.async_copy(xw_hbm.at[fidx_v.at[1]], r1, sem1)

            def body(j, carry):
                c0 = 2 * j
                pltpu.make_async_copy(xw_hbm.at[fidx_v.at[c0]], r0, sem0).wait()
                pltpu.sync_copy(r0, agg.at[dst_v.at[c0]], add=True)
                pltpu.async_copy(xw_hbm.at[fidx_v.at[c0 + 2]], r0, sem0)
                pltpu.make_async_copy(
                    xw_hbm.at[fidx_v.at[c0 + 1]], r1, sem1).wait()
                pltpu.sync_copy(r1, agg.at[dst_v.at[c0 + 1]], add=True)
                pltpu.async_copy(xw_hbm.at[fidx_v.at[c0 + 3]], r1, sem1)
                return carry

            lax.fori_loop(0, HALF // 2 - 1, body, 0)
            last = HALF - 2
            pltpu.make_async_copy(xw_hbm.at[fidx_v.at[last]], r0, sem0).wait()
            pltpu.sync_copy(r0, agg.at[dst_v.at[last]], add=True)
            pltpu.make_async_copy(xw_hbm.at[fidx_v.at[last + 1]], r1, sem1).wait()
            pltpu.sync_copy(r1, agg.at[dst_v.at[last + 1]], add=True)

        plsc.subcore_barrier()
        pltpu.sync_copy(agg.at[pl.ds(z0, rows)], out_hbm.at[c, pl.ds(z0, rows)])

    return edge_kernel


def _tail_body(p0_ref, p1_ref, x_ref, h1_ref, h2_ref, lw_ref, wg_ref,
               wih_ref, whh_ref, rb_ref, gb_ref, bih_ref, bhh_ref,
               h_ref, gate_ref, *, D):
    h1 = h1_ref[...]
    h2 = h2_ref[...]
    spatial = (p0_ref[0] + p1_ref[0]
               + jnp.dot(x_ref[...], lw_ref[...],
                         preferred_element_type=jnp.float32)
               + rb_ref[...])
    wg = wg_ref[...]
    gate_lin = (jnp.dot(spatial, wg[:D], preferred_element_type=jnp.float32)
                + jnp.dot(h1, wg[D:2 * D], preferred_element_type=jnp.float32)
                + jnp.dot(h2, wg[2 * D:], preferred_element_type=jnp.float32)
                + gb_ref[...])
    gate = jax.nn.sigmoid(gate_lin)
    fused = gate * h1 + (1.0 - gate) * h2
    gi = jnp.dot(spatial, wih_ref[...],
                 preferred_element_type=jnp.float32) + bih_ref[...]
    gh = jnp.dot(fused, whh_ref[...],
                 preferred_element_type=jnp.float32) + bhh_ref[...]
    r = jax.nn.sigmoid(gi[:, :D] + gh[:, :D])
    z = jax.nn.sigmoid(gi[:, D:2 * D] + gh[:, D:2 * D])
    n = jnp.tanh(gi[:, 2 * D:] + r * gh[:, 2 * D:])
    h_t = (1.0 - z) * n + z * fused
    h_ref[...] = jnp.maximum(h_t, 0.0)
    gate_ref[...] = gate


def kernel(x, prev_h_t1, prev_h_t2, edge_index, etype, rel_weight,
           loop_weight, rgcn_bias, gate_W, gate_b, W_ih, W_hh, b_ih, b_hh):
    N, D = x.shape
    R = rel_weight.shape[0]
    E = etype.shape[0]
    N_pad = ((N + 1) + NS * 8 - 1) // (NS * 8) * (NS * 8)
    E_pad = NW * NCHUNK * CHUNK
    pad = E_pad - E

    n_blk = N // ROW_BLK
    xw = pl.pallas_call(
        functools.partial(_table_matmul_body, R=R),
        grid=(n_blk,),
        in_specs=[pl.BlockSpec((ROW_BLK, D), lambda i: (i, 0)),
                  pl.BlockSpec((R, D, D), lambda i: (0, 0, 0))],
        out_specs=pl.BlockSpec((R, ROW_BLK, D), lambda i: (0, i, 0)),
        out_shape=jax.ShapeDtypeStruct((R, N, D), jnp.float32),
    )(x, rel_weight)

    fidx, dst_p = pl.pallas_call(
        functools.partial(_prep_body, N=N, R=R, N_pad=N_pad,
                          n_pad_rows=pad // 128),
        out_shape=[jax.ShapeDtypeStruct((E_pad // 128, 128), jnp.int32),
                   jax.ShapeDtypeStruct((E_pad // 128, 128), jnp.int32)],
    )(edge_index.reshape(2, E // 128, 128), etype.reshape(E // 128, 128))

    edge_kernel = _make_edge_kernel(N * R, D, N_pad)
    partials = edge_kernel(
        xw.reshape(R * N, D),
        fidx.reshape(NW, NCHUNK, CHUNK),
        dst_p.reshape(NW, NCHUNK, CHUNK),
    )

    wg_t = gate_W.T
    wih_t = W_ih.T
    whh_t = W_hh.T
    row = lambda i: (i, 0)
    fixed2 = lambda i: (0, 0)
    h_t, gate = pl.pallas_call(
        functools.partial(_tail_body, D=D),
        grid=(n_blk,),
        in_specs=[
            pl.BlockSpec((1, ROW_BLK, D), lambda i: (0, i, 0)),
            pl.BlockSpec((1, ROW_BLK, D), lambda i: (1, i, 0)),
            pl.BlockSpec((ROW_BLK, D), row),
            pl.BlockSpec((ROW_BLK, D), row),
            pl.BlockSpec((ROW_BLK, D), row),
            pl.BlockSpec((D, D), fixed2),
            pl.BlockSpec((3 * D, D), fixed2),
            pl.BlockSpec((D, 3 * D), fixed2),
            pl.BlockSpec((D, 3 * D), fixed2),
            pl.BlockSpec((1, D), fixed2),
            pl.BlockSpec((1, D), fixed2),
            pl.BlockSpec((1, 3 * D), fixed2),
            pl.BlockSpec((1, 3 * D), fixed2),
        ],
        out_specs=[pl.BlockSpec((ROW_BLK, D), row),
                   pl.BlockSpec((ROW_BLK, D), row)],
        out_shape=[jax.ShapeDtypeStruct((N, D), jnp.float32),
                   jax.ShapeDtypeStruct((N, D), jnp.float32)],
    )(partials, partials, x, prev_h_t1, prev_h_t2, loop_weight, wg_t,
      wih_t, whh_t, rgcn_bias.reshape(1, D), gate_b.reshape(1, D),
      b_ih.reshape(1, 3 * D), b_hh.reshape(1, 3 * D))
    return (h_t, gate)
